# baseline (speedup 1.0000x reference)
import jax
import jax.numpy as jnp
from jax import lax
from jax.experimental import pallas as pl
from jax.experimental.pallas import tpu as pltpu
from jax.experimental.pallas import tpu_sc as plsc

BATCH = 16384
HIDDEN = 32
NC, NS, L = 2, 16, 16
NW = NC * NS
BPW = BATCH // NW
GROUPS = BPW // L


def _pmf_body(user_ids, item_ids, emb_user, emb_item, out,
              uid_v, iid_v, u_rows, v_rows, out_v, sem_u, sem_v):
    wid = lax.axis_index("s") * NC + lax.axis_index("c")
    base = wid * BPW

    pltpu.sync_copy(user_ids.at[pl.ds(base, BPW)], uid_v)
    pltpu.sync_copy(item_ids.at[pl.ds(base, BPW)], iid_v)
    cp_u = pltpu.async_copy(emb_user.at[uid_v], u_rows, sem_u)
    cp_v = pltpu.async_copy(emb_item.at[iid_v], v_rows, sem_v)
    cp_u.wait()
    cp_v.wait()

    lanes = lax.iota(jnp.int32, L)

    def g_body(g, carry):
        rows = g * L + lanes
        acc = jnp.zeros((L,), jnp.float32)
        for h in range(HIDDEN):
            cols = jnp.full((L,), h, jnp.int32)
            gu = plsc.load_gather(u_rows, [rows, cols])
            gv = plsc.load_gather(v_rows, [rows, cols])
            acc = acc + gu * gv
        out_v[pl.ds(g * L, L)] = jnp.maximum(acc, 0.0)
        return carry

    lax.fori_loop(0, GROUPS, g_body, 0)
    pltpu.sync_copy(out_v, out.at[pl.ds(base, BPW)])


@jax.jit
def kernel(user_ids, item_ids, emb_user, emb_item):
    mesh = plsc.VectorSubcoreMesh(core_axis_name="c", subcore_axis_name="s",
                                  num_cores=NC, num_subcores=NS)
    k = pl.kernel(
        _pmf_body,
        out_type=jax.ShapeDtypeStruct((BATCH,), jnp.float32),
        mesh=mesh,
        scratch_types=[
            pltpu.VMEM((BPW,), jnp.int32),
            pltpu.VMEM((BPW,), jnp.int32),
            pltpu.VMEM((BPW, HIDDEN), jnp.float32),
            pltpu.VMEM((BPW, HIDDEN), jnp.float32),
            pltpu.VMEM((BPW,), jnp.float32),
            pltpu.SemaphoreType.DMA,
            pltpu.SemaphoreType.DMA,
        ],
        compiler_params=pltpu.CompilerParams(needs_layout_passes=False,
                                             use_tc_tiling_on_sc=False),
    )
    return k(user_ids.astype(jnp.int32), item_ids.astype(jnp.int32),
             emb_user, emb_item)
